# baseline (speedup 1.0000x reference)
import jax
import jax.numpy as jnp
from jax.experimental import pallas as pl
from jax.experimental.pallas import tpu as pltpu

_ACTION = 6
_HORIZON = 12
_ITERS = 2
_CAND = 1000
_TOP = 100
_HA = _HORIZON * _ACTION
_NCHUNK = 8
_CHUNK = 128
_H = 200
_Z = 30
_PAIR = 2
_ROWS = _PAIR * _CAND

_KEYS = ((0x4B665424, 0x9617674F), (0xAB7D1D1B, 0x652FBEF2))


def _gen_noise(lconst, g_off, k0, k1):
    u32 = jnp.uint32
    ks0 = u32(k0)
    ks1 = u32(k1)
    ks2 = u32(k0 ^ k1 ^ 0x1BD11BDA)
    ri = jax.lax.broadcasted_iota(u32, (_ROWS, _HA), 0)
    ctr = ri * u32(_ACTION) + lconst + g_off
    x0 = jnp.full((_ROWS, _HA), ks0, dtype=u32)
    x1 = ctr + ks1
    rots = ((13, 15, 26, 6), (17, 29, 16, 24))
    inj = ((ks1, ks2), (ks2, ks0), (ks0, ks1), (ks1, ks2), (ks2, ks0))
    for g in range(5):
        for r in rots[g % 2]:
            x0 = x0 + x1
            x1 = ((x1 << u32(r)) | (x1 >> u32(32 - r))) ^ x0
        ia, ib = inj[g]
        x0 = x0 + ia
        x1 = x1 + ib + u32(g + 1)
    bits = x0 ^ x1
    fl = jax.lax.bitcast_convert_type((bits >> u32(9)) | u32(0x3F800000),
                                      jnp.float32)
    lo = jnp.float32(-0.9999999403953552)
    rng = jnp.float32(1.0) - lo
    u = jnp.maximum(lo, (fl - jnp.float32(1.0)) * rng + lo)
    return jnp.float32(1.4142135381698608) * jax.lax.erf_inv(u)


def _planner_body(belief_ref, state_ref, WbZ_ref, WsE_ref, Wa_ref, wrs_ref,
                  lconst_ref, out_ref,
                  mean_s, std_s, rrow_s, noise0_s, noise1_s):
    WbZ = WbZ_ref[...]
    WsE = WsE_ref[...]
    Wa = Wa_ref[...]
    wrs = wrs_ref[...]
    lconst = lconst_ref[...]

    hp = jax.lax.Precision.HIGHEST
    g_off = (pl.program_id(0) * (_ROWS * _ACTION)).astype(jnp.uint32)

    ps0 = belief_ref[0, 0:1] @ WbZ[:, 0:_H] + state_ref[0, 0:1] @ WsE[:, 0:_H]
    ps1 = belief_ref[0, 1:2] @ WbZ[:, 0:_H] + state_ref[0, 1:2] @ WsE[:, 0:_H]
    row2 = jax.lax.broadcasted_iota(jnp.int32, (_ROWS, 1), 0)
    in_lo = row2 < _CAND
    pre_shared = jnp.where(in_lo, ps0, ps1)

    ii = jax.lax.broadcasted_iota(jnp.int32, (_CAND, _CHUNK), 0)
    jj = jax.lax.broadcasted_iota(jnp.int32, (_CAND, _CHUNK), 1)

    noise0_s[...] = _gen_noise(lconst, g_off, *_KEYS[0])
    noise1_s[...] = _gen_noise(lconst, g_off, *_KEYS[1])

    for it in range(_ITERS):
        noise_s = noise0_s if it == 0 else noise1_s

        if it == 0:
            mcat = None
            scat = None
        else:
            mcat = jnp.where(in_lo, mean_s[0:1], mean_s[1:2])
            scat = jnp.where(in_lo, std_s[0:1], std_s[1:2])

        if it == 0:
            a0 = noise_s[:, 0:_ACTION]
        else:
            a0 = (mcat[:, 0:_ACTION]
                  + scat[:, 0:_ACTION] * noise_s[:, 0:_ACTION])
        bc = jnp.tanh(pre_shared + a0 @ Wa)
        zz = bc @ WbZ
        sc = jnp.tanh(zz[:, _H:_H + _Z])
        bW = zz[:, 0:_H]
        rb = zz[:, _H + _Z:_H + _Z + 1]
        ret = None

        for h in range(1, _HORIZON):
            ss = sc @ WsE
            step_r = rb + ss[:, _H:_H + 1]
            ret = step_r if ret is None else ret + step_r
            c0 = _ACTION * h
            nh = noise_s[:, c0:c0 + _ACTION]
            if it == 0:
                a = nh
            else:
                a = mcat[:, c0:c0 + _ACTION] + scat[:, c0:c0 + _ACTION] * nh
            pre = bW + ss[:, 0:_H] + a @ Wa
            bc = jnp.tanh(pre)
            zz = bc @ WbZ
            sc = jnp.tanh(zz[:, _H:_H + _Z])
            bW = zz[:, 0:_H]
            rb = zz[:, _H + _Z:_H + _Z + 1]

        ret = ret + (rb + sc @ wrs)

        for half in range(_PAIR):
            r0 = half * _CAND
            reth = ret[r0:r0 + _CAND]

            rrow_s[...] = jnp.full((1, _NCHUNK * _CHUNK), -jnp.inf,
                                   dtype=jnp.float32)
            rrow_s[:, 0:_CAND] = reth.reshape(1, _CAND)

            rank = jnp.zeros((_CAND, 1), dtype=jnp.float32)
            for c in range(_NCHUNK):
                base = c * _CHUNK
                rj = rrow_s[:, base:base + _CHUNK]
                beats = (rj > reth) | ((rj == reth) & (jj + base < ii))
                rank = rank + jnp.sum(beats.astype(jnp.float32), axis=1,
                                      keepdims=True)
            maskT = (rank < float(_TOP)).astype(jnp.float32).reshape(1, _CAND)

            nch = noise_s[r0:r0 + _CAND]
            if it + 1 < _ITERS:
                ssum = jax.lax.dot(maskT, nch, precision=hp)
                mean_n = ssum * (1.0 / _TOP)
                cent = nch - mean_n
                msk2 = jax.lax.dot(maskT, cent * cent, precision=hp)
                std_n = jnp.sqrt(msk2 * (1.0 / _TOP))
                mean_s[half:half + 1] = mean_n
                std_s[half:half + 1] = std_n
            else:
                s6 = jax.lax.dot(maskT, nch[:, 0:_ACTION],
                                 precision=hp)
                out_ref[0, half] = (mean_s[half, 0:_ACTION]
                                    + std_s[half, 0:_ACTION]
                                    * (s6[0] * (1.0 / _TOP)))


def kernel(belief, state, Wb, Ws, Wa, Wz, wrb, wrs):
    B, H = belief.shape
    Z = state.shape[1]
    G = B // _PAIR

    WbZ = jnp.concatenate([Wb, Wz, wrb.reshape(H, 1)], axis=1)
    WsE = jnp.concatenate([Ws, wrs.reshape(Z, 1)], axis=1)

    cols = jnp.arange(_HA, dtype=jnp.uint32)
    lconst = ((cols // _ACTION) * jnp.uint32(B * _CAND * _ACTION)
              + cols % _ACTION).reshape(1, _HA)

    full = lambda *shape: pl.BlockSpec(shape, lambda b: (0,) * len(shape))

    out = pl.pallas_call(
        _planner_body,
        grid=(G,),
        in_specs=[
            pl.BlockSpec((1, _PAIR, H), lambda b: (b, 0, 0)),
            pl.BlockSpec((1, _PAIR, Z), lambda b: (b, 0, 0)),
            full(H, H + Z + 1),
            full(Z, H + 1),
            full(_ACTION, H),
            full(Z, 1),
            full(1, _HA),
        ],
        out_specs=pl.BlockSpec((1, _PAIR, _ACTION), lambda b: (b, 0, 0)),
        out_shape=jax.ShapeDtypeStruct((G, _PAIR, _ACTION), jnp.float32),
        scratch_shapes=[
            pltpu.VMEM((_PAIR, _HA), jnp.float32),
            pltpu.VMEM((_PAIR, _HA), jnp.float32),
            pltpu.VMEM((1, _NCHUNK * _CHUNK), jnp.float32),
            pltpu.VMEM((_ROWS, _HA), jnp.float32),
            pltpu.VMEM((_ROWS, _HA), jnp.float32),
        ],
    )(belief.reshape(G, _PAIR, H), state.reshape(G, _PAIR, Z), WbZ, WsE, Wa,
      wrs.reshape(Z, 1), lconst)
    return out.reshape(B, _ACTION)


# speedup vs baseline: 1.2185x; 1.2185x over previous
import jax
import jax.numpy as jnp
from jax.experimental import pallas as pl
from jax.experimental.pallas import tpu as pltpu

_ACTION = 6
_HORIZON = 12
_ITERS = 2
_CAND = 1000
_TOP = 100
_HA = _HORIZON * _ACTION
_NCHUNK = 8
_CHUNK = 128
_H = 200
_Z = 30
_B = 16

_KEYS = ((0x4B665424, 0x9617674F), (0xAB7D1D1B, 0x652FBEF2))


def _gen_noise(lconst, b_off, k0, k1):
    u32 = jnp.uint32
    ks0 = u32(k0)
    ks1 = u32(k1)
    ks2 = u32(k0 ^ k1 ^ 0x1BD11BDA)
    ri = jax.lax.broadcasted_iota(u32, (_CAND, _HA), 0)
    ctr = ri * u32(_ACTION) + lconst + b_off
    x0 = jnp.full((_CAND, _HA), ks0, dtype=u32)
    x1 = ctr + ks1
    rots = ((13, 15, 26, 6), (17, 29, 16, 24))
    inj = ((ks1, ks2), (ks2, ks0), (ks0, ks1), (ks1, ks2), (ks2, ks0))
    for g in range(5):
        for r in rots[g % 2]:
            x0 = x0 + x1
            x1 = ((x1 << u32(r)) | (x1 >> u32(32 - r))) ^ x0
        ia, ib = inj[g]
        x0 = x0 + ia
        x1 = x1 + ib + u32(g + 1)
    bits = x0 ^ x1
    fl = jax.lax.bitcast_convert_type((bits >> u32(9)) | u32(0x3F800000),
                                      jnp.float32)
    lo = jnp.float32(-0.9999999403953552)
    rng = jnp.float32(1.0) - lo
    u = jnp.maximum(lo, (fl - jnp.float32(1.0)) * rng + lo)
    return jnp.float32(1.4142135381698608) * jax.lax.erf_inv(u)


def _planner_body(belief_ref, state_ref, WbZ_ref, WsE_ref, Wa_ref, wrs_ref,
                  lconst_ref, out_ref,
                  mean_s, std_s, rrow_s, noise0_s, noise1_s):
    WbZ = WbZ_ref[...]
    WsE = WsE_ref[...]
    Wa = Wa_ref[...]
    wrs = wrs_ref[...]
    lconst = lconst_ref[...]

    hp = jax.lax.Precision.HIGHEST
    b_off = (pl.program_id(0) * (_CAND * _ACTION)).astype(jnp.uint32)

    pre_shared = (belief_ref[0] @ WbZ[:, 0:_H]
                  + state_ref[0] @ WsE[:, 0:_H])

    ii = jax.lax.broadcasted_iota(jnp.int32, (_CAND, _CHUNK), 0)
    jj = jax.lax.broadcasted_iota(jnp.int32, (_CAND, _CHUNK), 1)

    noise0_s[...] = _gen_noise(lconst, b_off, *_KEYS[0])
    noise1_s[...] = _gen_noise(lconst, b_off, *_KEYS[1])

    for it in range(_ITERS):
        noise_s = noise0_s if it == 0 else noise1_s

        if it == 0:
            a0 = noise_s[:, 0:_ACTION]
        else:
            a0 = mean_s[0] + std_s[0] * noise_s[:, 0:_ACTION]
        bc = jnp.tanh(pre_shared + a0 @ Wa)
        zz = bc @ WbZ
        sc = jnp.tanh(zz[:, _H:_H + _Z])
        bW = zz[:, 0:_H]
        rb = zz[:, _H + _Z:_H + _Z + 1]
        ret = None

        for h in range(1, _HORIZON):
            ss = sc @ WsE
            step_r = rb + ss[:, _H:_H + 1]
            ret = step_r if ret is None else ret + step_r
            nh = noise_s[:, _ACTION * h:_ACTION * (h + 1)]
            if it == 0:
                a = nh
            else:
                a = mean_s[h] + std_s[h] * nh
            pre = bW + ss[:, 0:_H] + a @ Wa
            bc = jnp.tanh(pre)
            zz = bc @ WbZ
            sc = jnp.tanh(zz[:, _H:_H + _Z])
            bW = zz[:, 0:_H]
            rb = zz[:, _H + _Z:_H + _Z + 1]

        ret = ret + (rb + sc @ wrs)

        rrow_s[...] = jnp.full((1, _NCHUNK * _CHUNK), -jnp.inf,
                               dtype=jnp.float32)
        rrow_s[:, 0:_CAND] = ret.reshape(1, _CAND)

        rank = jnp.zeros((_CAND, 1), dtype=jnp.float32)
        for c in range(_NCHUNK):
            base = c * _CHUNK
            rj = rrow_s[:, base:base + _CHUNK]
            beats = (rj > ret) | ((rj == ret) & (jj + base < ii))
            rank = rank + jnp.sum(beats.astype(jnp.float32), axis=1,
                                  keepdims=True)
        maskT = (rank < float(_TOP)).astype(jnp.float32).reshape(1, _CAND)

        if it + 1 < _ITERS:
            noise_cm = noise_s[...]
            ssum = jax.lax.dot(maskT, noise_cm, precision=hp)
            mean_n = ssum * (1.0 / _TOP)
            cent = noise_cm - mean_n
            msk2 = jax.lax.dot(maskT, cent * cent, precision=hp)
            std_n = jnp.sqrt(msk2 * (1.0 / _TOP))
            for h in range(_HORIZON):
                c0 = _ACTION * h
                mean_s[h] = mean_n[0, c0:c0 + _ACTION]
                std_s[h] = std_n[0, c0:c0 + _ACTION]
        else:
            s6 = jax.lax.dot(maskT, noise_s[:, 0:_ACTION],
                             precision=hp)
            out_ref[0, 0] = mean_s[0] + std_s[0] * (s6[0] * (1.0 / _TOP))


def kernel(belief, state, Wb, Ws, Wa, Wz, wrb, wrs):
    B, H = belief.shape
    Z = state.shape[1]

    WbZ = jnp.concatenate([Wb, Wz, wrb.reshape(H, 1)], axis=1)
    WsE = jnp.concatenate([Ws, wrs.reshape(Z, 1)], axis=1)

    cols = jnp.arange(_HA, dtype=jnp.uint32)
    lconst = ((cols // _ACTION) * jnp.uint32(B * _CAND * _ACTION)
              + cols % _ACTION).reshape(1, _HA)

    full = lambda *shape: pl.BlockSpec(shape, lambda b: (0,) * len(shape))

    out = pl.pallas_call(
        _planner_body,
        grid=(B,),
        in_specs=[
            pl.BlockSpec((1, 1, H), lambda b: (b, 0, 0)),
            pl.BlockSpec((1, 1, Z), lambda b: (b, 0, 0)),
            full(H, H + Z + 1),
            full(Z, H + 1),
            full(_ACTION, H),
            full(Z, 1),
            full(1, _HA),
        ],
        out_specs=pl.BlockSpec((1, 1, _ACTION), lambda b: (b, 0, 0)),
        out_shape=jax.ShapeDtypeStruct((B, 1, _ACTION), jnp.float32),
        scratch_shapes=[
            pltpu.VMEM((_HORIZON, _ACTION), jnp.float32),
            pltpu.VMEM((_HORIZON, _ACTION), jnp.float32),
            pltpu.VMEM((1, _NCHUNK * _CHUNK), jnp.float32),
            pltpu.VMEM((_CAND, _HA), jnp.float32),
            pltpu.VMEM((_CAND, _HA), jnp.float32),
        ],
    )(belief.reshape(B, 1, H), state.reshape(B, 1, Z), WbZ, WsE, Wa,
      wrs.reshape(Z, 1), lconst)
    return out.reshape(B, _ACTION)
